# Initial kernel scaffold; baseline (speedup 1.0000x reference)
import functools

import jax
import jax.numpy as jnp
from jax.experimental import pallas as pl

B, S, D, OUT, E = 2, 4096, 768, 768, 8
TILE = 1024


def _moe_ffn_kernel(x_ref, wg_ref, bg_ref, wa_ref, ba_ref, wb_ref, bb_ref,
                    wc_ref, bc_ref, o_ref):
    x = x_ref[...]
    logits = jnp.dot(x, wg_ref[...], preferred_element_type=jnp.float32)
    logits = logits + bg_ref[...]
    g = jax.nn.softmax(logits, axis=-1)
    acc = jnp.dot(g, bc_ref[...], preferred_element_type=jnp.float32)
    for e in range(E):
        a = jnp.dot(x, wa_ref[e], preferred_element_type=jnp.float32)
        a = a + ba_ref[e]
        b = jnp.dot(x, wb_ref[e], preferred_element_type=jnp.float32)
        b = b + bb_ref[e]
        h = (a * jax.nn.sigmoid(a)) * b
        hg = h * g[:, e:e + 1]
        acc = acc + jnp.dot(hg, wc_ref[e], preferred_element_type=jnp.float32)
    o_ref[...] = acc


@functools.partial(jax.jit, static_argnames=("interpret",))
def _moe_ffn(x, Wg, bg, Wa, ba, Wb, bb, Wc, bc, interpret=False):
    n = x.shape[0]
    grid = (n // TILE,)
    const = lambda ndim: pl.BlockSpec(None, lambda i: (0,) * ndim)
    return pl.pallas_call(
        _moe_ffn_kernel,
        grid=grid,
        in_specs=[
            pl.BlockSpec((TILE, D), lambda i: (i, 0)),
            const(2),
            const(2),
            const(3),
            const(2),
            const(3),
            const(2),
            const(3),
            const(2),
        ],
        out_specs=pl.BlockSpec((TILE, OUT), lambda i: (i, 0)),
        out_shape=jax.ShapeDtypeStruct((n, OUT), jnp.float32),
        interpret=interpret,
    )(x, Wg, bg, Wa, ba, Wb, bb, Wc, bc)


def kernel(inputs, Wg, bg, Wa, ba, Wb, bb, Wc, bc):
    b, s, d = inputs.shape
    x = inputs.reshape(b * s, d)
    out = _moe_ffn(x, Wg, bg.reshape(1, E), Wa, ba, Wb, bb, Wc, bc)
    return out.reshape(b, s, OUT)


# speedup vs baseline: 2.1814x; 2.1814x over previous
import functools

import jax
import jax.numpy as jnp
from jax.experimental import pallas as pl
from jax.experimental.pallas import tpu as pltpu

B, S, D, OUT, E = 2, 4096, 768, 768, 8
TILE = 1024


def _moe_ffn_kernel(x_ref, wg_ref, bg_ref, wa_ref, ba_ref, wb_ref, bb_ref,
                    wc_ref, bc_ref, o_ref, g_scratch):
    e = pl.program_id(1)
    x = x_ref[...]

    @pl.when(e == 0)
    def _init():
        logits = jnp.dot(x, wg_ref[...], preferred_element_type=jnp.float32)
        logits = logits + bg_ref[...]
        g = jax.nn.softmax(logits, axis=-1)
        g_scratch[...] = g
        o_ref[...] = jnp.dot(g, bc_ref[...], preferred_element_type=jnp.float32)

    gates = g_scratch[...]
    lane = jax.lax.broadcasted_iota(jnp.int32, gates.shape, 1)
    g_e = jnp.sum(jnp.where(lane == e, gates, 0.0), axis=1, keepdims=True)

    a = jnp.dot(x, wa_ref[0], preferred_element_type=jnp.float32) + ba_ref[0]
    b = jnp.dot(x, wb_ref[0], preferred_element_type=jnp.float32) + bb_ref[0]
    h = (a * jax.lax.logistic(a)) * b
    o_ref[...] += jnp.dot(h * g_e, wc_ref[0],
                          preferred_element_type=jnp.float32)


@jax.jit
def _moe_ffn(x, Wg, bg, Wa, ba, Wb, bb, Wc, bc):
    n = x.shape[0]
    grid = (n // TILE, E)
    return pl.pallas_call(
        _moe_ffn_kernel,
        grid=grid,
        in_specs=[
            pl.BlockSpec((TILE, D), lambda i, e: (i, 0)),
            pl.BlockSpec((D, E), lambda i, e: (0, 0)),
            pl.BlockSpec((1, E), lambda i, e: (0, 0)),
            pl.BlockSpec((1, D, OUT), lambda i, e: (e, 0, 0)),
            pl.BlockSpec((1, 1, OUT), lambda i, e: (e, 0, 0)),
            pl.BlockSpec((1, D, OUT), lambda i, e: (e, 0, 0)),
            pl.BlockSpec((1, 1, OUT), lambda i, e: (e, 0, 0)),
            pl.BlockSpec((1, OUT, OUT), lambda i, e: (e, 0, 0)),
            pl.BlockSpec((E, OUT), lambda i, e: (0, 0)),
        ],
        out_specs=pl.BlockSpec((TILE, OUT), lambda i, e: (i, 0)),
        out_shape=jax.ShapeDtypeStruct((n, OUT), jnp.float32),
        scratch_shapes=[pltpu.VMEM((TILE, E), jnp.float32)],
    )(x, Wg, bg, Wa, ba, Wb, bb, Wc, bc)


def kernel(inputs, Wg, bg, Wa, ba, Wb, bb, Wc, bc):
    b, s, d = inputs.shape
    x = inputs.reshape(b * s, d)
    out = _moe_ffn(x, Wg, bg.reshape(1, E), Wa, ba.reshape(E, 1, OUT), Wb,
                   bb.reshape(E, 1, OUT), Wc, bc)
    return out.reshape(b, s, OUT)


# speedup vs baseline: 2.1951x; 1.0063x over previous
import functools

import jax
import jax.numpy as jnp
from jax.experimental import pallas as pl
from jax.experimental.pallas import tpu as pltpu

B, S, D, OUT, E = 2, 4096, 768, 768, 8
TILE = 2048


def _moe_ffn_kernel(x_ref, wg_ref, bg_ref, wa_ref, ba_ref, wb_ref, bb_ref,
                    wc_ref, bc_ref, o_ref, g_scratch):
    e = pl.program_id(1)
    x = x_ref[...]

    @pl.when(e == 0)
    def _init():
        logits = jnp.dot(x, wg_ref[...], preferred_element_type=jnp.float32)
        logits = logits + bg_ref[...]
        g = jax.nn.softmax(logits, axis=-1)
        g_scratch[...] = g
        o_ref[...] = jnp.dot(g, bc_ref[...], preferred_element_type=jnp.float32)

    gates = g_scratch[...]
    lane = jax.lax.broadcasted_iota(jnp.int32, gates.shape, 1)
    g_e = jnp.sum(jnp.where(lane == e, gates, 0.0), axis=1, keepdims=True)

    a = jnp.dot(x, wa_ref[0], preferred_element_type=jnp.float32) + ba_ref[0]
    b = jnp.dot(x, wb_ref[0], preferred_element_type=jnp.float32) + bb_ref[0]
    h = (a * jax.lax.logistic(a)) * b
    o_ref[...] += jnp.dot(h * g_e, wc_ref[0],
                          preferred_element_type=jnp.float32)


@jax.jit
def _moe_ffn(x, Wg, bg, Wa, ba, Wb, bb, Wc, bc):
    n = x.shape[0]
    grid = (n // TILE, E)
    return pl.pallas_call(
        _moe_ffn_kernel,
        grid=grid,
        in_specs=[
            pl.BlockSpec((TILE, D), lambda i, e: (i, 0)),
            pl.BlockSpec((D, E), lambda i, e: (0, 0)),
            pl.BlockSpec((1, E), lambda i, e: (0, 0)),
            pl.BlockSpec((1, D, OUT), lambda i, e: (e, 0, 0)),
            pl.BlockSpec((1, 1, OUT), lambda i, e: (e, 0, 0)),
            pl.BlockSpec((1, D, OUT), lambda i, e: (e, 0, 0)),
            pl.BlockSpec((1, 1, OUT), lambda i, e: (e, 0, 0)),
            pl.BlockSpec((1, OUT, OUT), lambda i, e: (e, 0, 0)),
            pl.BlockSpec((E, OUT), lambda i, e: (0, 0)),
        ],
        out_specs=pl.BlockSpec((TILE, OUT), lambda i, e: (i, 0)),
        out_shape=jax.ShapeDtypeStruct((n, OUT), jnp.float32),
        scratch_shapes=[pltpu.VMEM((TILE, E), jnp.float32)],
    )(x, Wg, bg, Wa, ba, Wb, bb, Wc, bc)


def kernel(inputs, Wg, bg, Wa, ba, Wb, bb, Wc, bc):
    b, s, d = inputs.shape
    x = inputs.reshape(b * s, d)
    out = _moe_ffn(x, Wg, bg.reshape(1, E), Wa, ba.reshape(E, 1, OUT), Wb,
                   bb.reshape(E, 1, OUT), Wc, bc)
    return out.reshape(b, s, OUT)
